# baseline (device time: 19036 ns/iter reference)
import jax
import jax.numpy as jnp
from jax import lax
from jax.experimental import pallas as pl
from jax.experimental.pallas import tpu as pltpu

N_DEV = 8
B = 2
SQ = 256
HALO = 128
HQ = 4
DH = 64
DM = 512


def kernel(x, Wq, K_ext, V_ext, Wo):
    Kt = jnp.transpose(K_ext, (0, 2, 1, 3)).astype(jnp.bfloat16)
    Vt = jnp.transpose(V_ext, (0, 2, 1, 3)).astype(jnp.bfloat16)
    Wqt = (jnp.transpose(Wq.reshape(DM, HQ, DH), (1, 0, 2)) * 0.125
           ).astype(jnp.bfloat16)
    x16 = x.astype(jnp.bfloat16)
    Wo16 = Wo.astype(jnp.bfloat16)

    def body(x_ref, wqt_ref, k_ref, v_ref, wo_ref, out_ref,
             klbuf, vlbuf, krbuf, vrbuf, send_sems, recv_sems):
        my = lax.axis_index("i")
        left = jnp.maximum(my - 1, 0)
        right = jnp.minimum(my + 1, N_DEV - 1)

        barrier_sem = pltpu.get_barrier_semaphore()

        @pl.when(my > 0)
        def _():
            pl.semaphore_signal(barrier_sem, inc=1, device_id=(left,),
                                device_id_type=pltpu.DeviceIdType.MESH)

        @pl.when(my < N_DEV - 1)
        def _():
            pl.semaphore_signal(barrier_sem, inc=1, device_id=(right,),
                                device_id_type=pltpu.DeviceIdType.MESH)

        @pl.when(my == 0)
        def _():
            vlbuf[...] = jnp.zeros((B, HQ, HALO, DH), jnp.bfloat16)

        @pl.when(my == N_DEV - 1)
        def _():
            vrbuf[...] = jnp.zeros((B, HQ, HALO, DH), jnp.bfloat16)

        n_nbrs = (my > 0).astype(jnp.int32) + (my < N_DEV - 1).astype(jnp.int32)
        pl.semaphore_wait(barrier_sem, n_nbrs)

        rdma_r_k = pltpu.make_async_remote_copy(
            src_ref=k_ref.at[:, :, pl.ds(SQ - HALO, HALO)],
            dst_ref=klbuf,
            send_sem=send_sems.at[0], recv_sem=recv_sems.at[0],
            device_id=(right,), device_id_type=pltpu.DeviceIdType.MESH,
        )
        rdma_r_v = pltpu.make_async_remote_copy(
            src_ref=v_ref.at[:, :, pl.ds(SQ - HALO, HALO)],
            dst_ref=vlbuf,
            send_sem=send_sems.at[1], recv_sem=recv_sems.at[1],
            device_id=(right,), device_id_type=pltpu.DeviceIdType.MESH,
        )
        rdma_l_k = pltpu.make_async_remote_copy(
            src_ref=k_ref.at[:, :, pl.ds(0, HALO)],
            dst_ref=krbuf,
            send_sem=send_sems.at[2], recv_sem=recv_sems.at[2],
            device_id=(left,), device_id_type=pltpu.DeviceIdType.MESH,
        )
        rdma_l_v = pltpu.make_async_remote_copy(
            src_ref=v_ref.at[:, :, pl.ds(0, HALO)],
            dst_ref=vrbuf,
            send_sem=send_sems.at[3], recv_sem=recv_sems.at[3],
            device_id=(left,), device_id_type=pltpu.DeviceIdType.MESH,
        )

        @pl.when(my < N_DEV - 1)
        def _():
            rdma_r_k.start()
            rdma_r_v.start()

        @pl.when(my > 0)
        def _():
            rdma_l_k.start()
            rdma_l_v.start()

        r_a = lax.broadcasted_iota(jnp.int32, (SQ, SQ), 0)
        j_a = lax.broadcasted_iota(jnp.int32, (SQ, SQ), 1)
        mask_a = jnp.abs(j_a - r_a) <= HALO

        q = []
        ctx = []
        lsum = []
        for b in range(B):
            q.append([])
            ctx.append([])
            lsum.append([])
            for h in range(HQ):
                qh = jnp.dot(x_ref[b], wqt_ref[h],
                             preferred_element_type=jnp.float32)
                qh16 = qh.astype(jnp.bfloat16)
                s = lax.dot_general(
                    qh16, k_ref[b, h], (((1,), (1,)), ((), ())),
                    preferred_element_type=jnp.float32,
                )
                w = jnp.where(mask_a, jnp.exp(s), 0.0)
                ctx_h = jnp.dot(w.astype(jnp.bfloat16), v_ref[b, h],
                                preferred_element_type=jnp.float32)
                q[b].append(qh16)
                ctx[b].append(ctx_h)
                lsum[b].append(jnp.sum(w, axis=1, keepdims=True))

        @pl.when(my > 0)
        def _():
            rdma_r_k.wait_recv()

        @pl.when(my < N_DEV - 1)
        def _():
            rdma_l_k.wait_recv()

        r_h = lax.broadcasted_iota(jnp.int32, (SQ, HALO), 0)
        j_h = lax.broadcasted_iota(jnp.int32, (SQ, HALO), 1)
        mask_l = (j_h >= r_h) & (my > 0)
        mask_r = (j_h <= r_h - HALO) & (my < N_DEV - 1)

        wl = []
        wr = []
        for b in range(B):
            wl.append([])
            wr.append([])
            for h in range(HQ):
                s_l = lax.dot_general(
                    q[b][h], klbuf[b, h], (((1,), (1,)), ((), ())),
                    preferred_element_type=jnp.float32,
                )
                s_r = lax.dot_general(
                    q[b][h], krbuf[b, h], (((1,), (1,)), ((), ())),
                    preferred_element_type=jnp.float32,
                )
                wl[b].append(jnp.where(mask_l, jnp.exp(s_l), 0.0))
                wr[b].append(jnp.where(mask_r, jnp.exp(s_r), 0.0))

        @pl.when(my > 0)
        def _():
            rdma_r_v.wait_recv()

        @pl.when(my < N_DEV - 1)
        def _():
            rdma_l_v.wait_recv()

        @pl.when(my < N_DEV - 1)
        def _():
            rdma_r_k.wait_send()
            rdma_r_v.wait_send()

        @pl.when(my > 0)
        def _():
            rdma_l_k.wait_send()
            rdma_l_v.wait_send()

        for b in range(B):
            acc = jnp.zeros((SQ, DM), jnp.float32)
            for h in range(HQ):
                w_l = wl[b][h]
                w_r = wr[b][h]
                ctx_h = (
                    ctx[b][h]
                    + jnp.dot(w_l.astype(jnp.bfloat16), vlbuf[b, h],
                              preferred_element_type=jnp.float32)
                    + jnp.dot(w_r.astype(jnp.bfloat16), vrbuf[b, h],
                              preferred_element_type=jnp.float32)
                )
                l_h = (lsum[b][h]
                       + jnp.sum(w_l, axis=1, keepdims=True)
                       + jnp.sum(w_r, axis=1, keepdims=True))
                ctx_h = ctx_h / l_h
                acc += jnp.dot(ctx_h.astype(jnp.bfloat16), wo_ref[h * DH:(h + 1) * DH, :],
                               preferred_element_type=jnp.float32)
            out_ref[b] = acc

    halo_shape = (B, HQ, HALO, DH)
    return pl.pallas_call(
        body,
        out_shape=jax.ShapeDtypeStruct(x.shape, jnp.float32),
        in_specs=[pl.BlockSpec(memory_space=pltpu.VMEM)] * 5,
        out_specs=pl.BlockSpec(memory_space=pltpu.VMEM),
        compiler_params=pltpu.CompilerParams(collective_id=0),
        scratch_shapes=[
            pltpu.VMEM(halo_shape, jnp.bfloat16),
            pltpu.VMEM(halo_shape, jnp.bfloat16),
            pltpu.VMEM(halo_shape, jnp.bfloat16),
            pltpu.VMEM(halo_shape, jnp.bfloat16),
            pltpu.SemaphoreType.DMA((4,)),
            pltpu.SemaphoreType.DMA((4,)),
        ],
    )(x16, Wqt, Kt, Vt, Wo16)


# device time: 18752 ns/iter; 1.0151x vs baseline; 1.0151x over previous
import jax
import jax.numpy as jnp
from jax import lax
from jax.experimental import pallas as pl
from jax.experimental.pallas import tpu as pltpu

N_DEV = 8
B = 2
SQ = 256
HALO = 128
HQ = 4
DH = 64
DM = 512


def kernel(x, Wq, K_ext, V_ext, Wo):
    Kt = jnp.transpose(K_ext, (0, 2, 1, 3)).astype(jnp.bfloat16)
    Vt = jnp.transpose(V_ext, (0, 2, 1, 3)).astype(jnp.bfloat16)
    Wqt = (jnp.transpose(Wq.reshape(DM, HQ, DH), (1, 0, 2)) * 0.125
           ).astype(jnp.bfloat16)
    Wo16 = Wo.astype(jnp.bfloat16)

    def body(x_ref, wqt_ref, k_ref, v_ref, wo_ref, out_ref,
             khalo, vhalo, send_sems, recv_sems):
        my = lax.axis_index("i")
        left = jnp.maximum(my - 1, 0)
        right = jnp.minimum(my + 1, N_DEV - 1)

        barrier_sem = pltpu.get_barrier_semaphore()

        @pl.when(my > 0)
        def _():
            pl.semaphore_signal(barrier_sem, inc=1, device_id=(left,),
                                device_id_type=pltpu.DeviceIdType.MESH)

        @pl.when(my < N_DEV - 1)
        def _():
            pl.semaphore_signal(barrier_sem, inc=1, device_id=(right,),
                                device_id_type=pltpu.DeviceIdType.MESH)

        @pl.when(my == 0)
        def _():
            vhalo[:, :, 0:HALO] = jnp.zeros((B, HQ, HALO, DH), jnp.bfloat16)

        @pl.when(my == N_DEV - 1)
        def _():
            vhalo[:, :, HALO:2 * HALO] = jnp.zeros((B, HQ, HALO, DH),
                                                   jnp.bfloat16)

        n_nbrs = (my > 0).astype(jnp.int32) + (my < N_DEV - 1).astype(jnp.int32)
        pl.semaphore_wait(barrier_sem, n_nbrs)

        rdma_r_k = pltpu.make_async_remote_copy(
            src_ref=k_ref.at[:, :, pl.ds(SQ - HALO, HALO)],
            dst_ref=khalo.at[:, :, pl.ds(0, HALO)],
            send_sem=send_sems.at[0], recv_sem=recv_sems.at[0],
            device_id=(right,), device_id_type=pltpu.DeviceIdType.MESH,
        )
        rdma_l_k = pltpu.make_async_remote_copy(
            src_ref=k_ref.at[:, :, pl.ds(0, HALO)],
            dst_ref=khalo.at[:, :, pl.ds(HALO, HALO)],
            send_sem=send_sems.at[1], recv_sem=recv_sems.at[1],
            device_id=(left,), device_id_type=pltpu.DeviceIdType.MESH,
        )
        rdma_r_v = pltpu.make_async_remote_copy(
            src_ref=v_ref.at[:, :, pl.ds(SQ - HALO, HALO)],
            dst_ref=vhalo.at[:, :, pl.ds(0, HALO)],
            send_sem=send_sems.at[2], recv_sem=recv_sems.at[2],
            device_id=(right,), device_id_type=pltpu.DeviceIdType.MESH,
        )
        rdma_l_v = pltpu.make_async_remote_copy(
            src_ref=v_ref.at[:, :, pl.ds(0, HALO)],
            dst_ref=vhalo.at[:, :, pl.ds(HALO, HALO)],
            send_sem=send_sems.at[3], recv_sem=recv_sems.at[3],
            device_id=(left,), device_id_type=pltpu.DeviceIdType.MESH,
        )

        @pl.when(my < N_DEV - 1)
        def _():
            rdma_r_k.start()

        @pl.when(my > 0)
        def _():
            rdma_l_k.start()

        @pl.when(my < N_DEV - 1)
        def _():
            rdma_r_v.start()

        @pl.when(my > 0)
        def _():
            rdma_l_v.start()

        r_a = lax.broadcasted_iota(jnp.int32, (SQ, SQ), 0)
        j_a = lax.broadcasted_iota(jnp.int32, (SQ, SQ), 1)
        mask_a = jnp.abs(j_a - r_a) <= HALO

        q = []
        ctx = []
        lsum = []
        for b in range(B):
            xb16 = x_ref[b].astype(jnp.bfloat16)
            q.append([])
            ctx.append([])
            lsum.append([])
            for h in range(HQ):
                qh = jnp.dot(xb16, wqt_ref[h],
                             preferred_element_type=jnp.float32)
                qh16 = qh.astype(jnp.bfloat16)
                s = lax.dot_general(
                    qh16, k_ref[b, h], (((1,), (1,)), ((), ())),
                    preferred_element_type=jnp.float32,
                )
                w = jnp.where(mask_a, jnp.exp(s), 0.0)
                ctx_h = jnp.dot(w.astype(jnp.bfloat16), v_ref[b, h],
                                preferred_element_type=jnp.float32)
                q[b].append(qh16)
                ctx[b].append(ctx_h)
                lsum[b].append(jnp.sum(w, axis=1, keepdims=True))

        @pl.when(my > 0)
        def _():
            rdma_r_k.wait_recv()

        @pl.when(my < N_DEV - 1)
        def _():
            rdma_l_k.wait_recv()

        r_h = lax.broadcasted_iota(jnp.int32, (SQ, 2 * HALO), 0)
        j_h = lax.broadcasted_iota(jnp.int32, (SQ, 2 * HALO), 1)
        mask_h = ((j_h < HALO) & (j_h >= r_h) & (my > 0)) | (
            (j_h >= HALO) & (j_h <= r_h) & (my < N_DEV - 1))

        wh = []
        for b in range(B):
            wh.append([])
            for h in range(HQ):
                s_h = lax.dot_general(
                    q[b][h], khalo[b, h], (((1,), (1,)), ((), ())),
                    preferred_element_type=jnp.float32,
                )
                wh[b].append(jnp.where(mask_h, jnp.exp(s_h), 0.0))

        @pl.when(my > 0)
        def _():
            rdma_r_v.wait_recv()

        @pl.when(my < N_DEV - 1)
        def _():
            rdma_l_v.wait_recv()

        @pl.when(my < N_DEV - 1)
        def _():
            rdma_r_k.wait_send()
            rdma_r_v.wait_send()

        @pl.when(my > 0)
        def _():
            rdma_l_k.wait_send()
            rdma_l_v.wait_send()

        for b in range(B):
            acc = jnp.zeros((SQ, DM), jnp.float32)
            for h in range(HQ):
                w_h = wh[b][h]
                ctx_h = ctx[b][h] + jnp.dot(
                    w_h.astype(jnp.bfloat16), vhalo[b, h],
                    preferred_element_type=jnp.float32,
                )
                l_h = lsum[b][h] + jnp.sum(w_h, axis=1, keepdims=True)
                ctx_h = ctx_h / l_h
                acc += jnp.dot(ctx_h.astype(jnp.bfloat16),
                               wo_ref[h * DH:(h + 1) * DH, :],
                               preferred_element_type=jnp.float32)
            out_ref[b] = acc

    return pl.pallas_call(
        body,
        out_shape=jax.ShapeDtypeStruct(x.shape, jnp.float32),
        in_specs=[pl.BlockSpec(memory_space=pltpu.VMEM)] * 5,
        out_specs=pl.BlockSpec(memory_space=pltpu.VMEM),
        compiler_params=pltpu.CompilerParams(collective_id=0),
        scratch_shapes=[
            pltpu.VMEM((B, HQ, 2 * HALO, DH), jnp.bfloat16),
            pltpu.VMEM((B, HQ, 2 * HALO, DH), jnp.bfloat16),
            pltpu.SemaphoreType.DMA((4,)),
            pltpu.SemaphoreType.DMA((4,)),
        ],
    )(x, Wqt, Kt, Vt, Wo16)
